# Initial kernel scaffold; baseline (speedup 1.0000x reference)
import functools

import jax
import jax.numpy as jnp
from jax import lax
from jax.experimental import pallas as pl
from jax.experimental.pallas import tpu as pltpu
from jax.experimental.pallas import tpu_sc as plsc

N = 10000
D = 128
C = 32

NC = 2
NS = 16
NW = NC * NS
LANES = 16

NPAD = 10240
E = 160000
EPAD = 163840
ECH = 128
EPT = EPAD // NW
NCH = EPT // ECH
ROWS_PT = NPAD // NS
DEGW = 16

_mesh = functools.partial(plsc.VectorSubcoreMesh,
                          core_axis_name="c", subcore_axis_name="s")


def _sc_edge_pass(F):
    @functools.partial(
        pl.kernel,
        out_type=jax.ShapeDtypeStruct((NC, NPAD, F), jnp.float32),
        mesh=_mesh(),
        scratch_types=[
            pltpu.VMEM((NCH, ECH), jnp.int32),
            pltpu.VMEM((NCH, ECH), jnp.int32),
            pltpu.VMEM((NCH, ECH), jnp.float32),
            pltpu.VMEM((ECH, F), jnp.float32),
            pltpu.VMEM_SHARED((NPAD, F), jnp.float32),
            pltpu.SemaphoreType.DMA,
        ],
    )
    def k(y_hbm, src_hbm, dst_hbm, ew_hbm, out_hbm,
          src_v, dst_v, ew_v, rows_v, acc_sh, sem):
        cid = lax.axis_index("c")
        sid = lax.axis_index("s")
        t = cid * NS + sid
        ebase = t * NCH
        pltpu.sync_copy(src_hbm.at[pl.ds(ebase, NCH)], src_v)
        pltpu.sync_copy(dst_hbm.at[pl.ds(ebase, NCH)], dst_v)
        pltpu.sync_copy(ew_hbm.at[pl.ds(ebase, NCH)], ew_v)

        zv = jnp.zeros((LANES,), jnp.float32)

        def zrow(r, carry):
            for cb in range(F // LANES):
                rows_v[r, pl.ds(cb * LANES, LANES)] = zv
            return carry

        lax.fori_loop(0, ECH, zrow, 0)
        rbase = sid * ROWS_PT
        for rb in range(ROWS_PT // ECH):
            pltpu.sync_copy(rows_v, acc_sh.at[pl.ds(rbase + rb * ECH, ECH)])
        plsc.subcore_barrier()

        def chunk(j, carry):
            pltpu.async_copy(y_hbm.at[src_v.at[j]], rows_v, sem).wait()

            def grp(kk, c2):
                w16 = ew_v[j, pl.ds(kk * LANES, LANES)]
                for l in range(LANES):
                    w = jnp.broadcast_to(w16[l], (LANES,))
                    e = kk * LANES + l
                    for cb in range(F // LANES):
                        sl = pl.ds(cb * LANES, LANES)
                        rows_v[e, sl] = rows_v[e, sl] * w
                return c2

            lax.fori_loop(0, ECH // LANES, grp, 0)
            pltpu.sync_copy(rows_v, acc_sh.at[dst_v.at[j]], add=True)
            return carry

        lax.fori_loop(0, NCH, chunk, 0)
        plsc.subcore_barrier()

        for rb in range(ROWS_PT // ECH):
            sl = pl.ds(rbase + rb * ECH, ECH)
            pltpu.sync_copy(acc_sh.at[sl], rows_v)
            pltpu.sync_copy(rows_v, out_hbm.at[cid, sl])

    return k


@functools.partial(
    pl.kernel,
    out_type=jax.ShapeDtypeStruct((NC, NPAD, DEGW), jnp.float32),
    mesh=_mesh(),
    scratch_types=[
        pltpu.VMEM((NCH, ECH), jnp.int32),
        pltpu.VMEM((ECH, DEGW), jnp.float32),
        pltpu.VMEM_SHARED((NPAD, DEGW), jnp.float32),
    ],
)
def _sc_degree(ew16_hbm, dst_hbm, out_hbm, dst_v, rows_v, acc_sh):
    cid = lax.axis_index("c")
    sid = lax.axis_index("s")
    t = cid * NS + sid
    pltpu.sync_copy(dst_hbm.at[pl.ds(t * NCH, NCH)], dst_v)

    zv = jnp.zeros((LANES,), jnp.float32)

    def zrow(r, carry):
        rows_v[r, pl.ds(0, LANES)] = zv
        return carry

    lax.fori_loop(0, ECH, zrow, 0)
    rbase = sid * ROWS_PT
    for rb in range(ROWS_PT // ECH):
        pltpu.sync_copy(rows_v, acc_sh.at[pl.ds(rbase + rb * ECH, ECH)])
    plsc.subcore_barrier()

    def chunk(j, carry):
        pltpu.sync_copy(ew16_hbm.at[pl.ds(t * EPT + j * ECH, ECH)], rows_v)
        pltpu.sync_copy(rows_v, acc_sh.at[dst_v.at[j]], add=True)
        return carry

    lax.fori_loop(0, NCH, chunk, 0)
    plsc.subcore_barrier()

    for rb in range(ROWS_PT // ECH):
        sl = pl.ds(rbase + rb * ECH, ECH)
        pltpu.sync_copy(acc_sh.at[sl], rows_v)
        pltpu.sync_copy(rows_v, out_hbm.at[cid, sl])


RB = 1280


def _deg_stats(degp_blk):
    deg = 1.0 + degp_blk[0, :, 0] + degp_blk[1, :, 0]
    return lax.rsqrt(deg), 1.0 / deg


def _tc_xw_y(x_ref, w_ref, degp_ref, xw_ref, y_ref):
    dis, _ = _deg_stats(degp_ref[...])
    xw = jnp.dot(x_ref[...], w_ref[...], preferred_element_type=jnp.float32)
    xw_ref[...] = xw
    y_ref[...] = xw * dis[:, None]


def _tc_mid(a_ref, xw1_ref, degp_ref, b1_ref, w2_ref, xw2_ref, y2_ref):
    dis, inv = _deg_stats(degp_ref[...])
    a = a_ref[...]
    h = dis[:, None] * (a[0] + a[1]) + inv[:, None] * xw1_ref[...] + b1_ref[...]
    h = jnp.maximum(h, 0.0)
    xw2 = jnp.dot(h, w2_ref[...], preferred_element_type=jnp.float32)
    xw2_ref[...] = xw2
    y2_ref[...] = xw2 * dis[:, None]


def _tc_final(a_ref, xw2_ref, degp_ref, b2_ref, o_ref):
    dis, inv = _deg_stats(degp_ref[...])
    a = a_ref[...]
    s = dis[:, None] * (a[0] + a[1]) + inv[:, None] * xw2_ref[...] + b2_ref[...]
    m = jnp.max(s, axis=-1, keepdims=True)
    ex = jnp.exp(s - m)
    o_ref[...] = ex / jnp.sum(ex, axis=-1, keepdims=True)


def _row_spec(f):
    return pl.BlockSpec((RB, f), lambda i: (i, 0))


def _part_spec(f):
    return pl.BlockSpec((NC, RB, f), lambda i: (0, i, 0))


def _full_spec(r, f):
    return pl.BlockSpec((r, f), lambda i: (0, 0))


_GRID = NPAD // RB


def kernel(x, edge_index, edge_weight, gat_W, gat_att_src, gat_att_dst,
           gat_bias, gcn1_W, gcn1_b, gcn2_W, gcn2_b):
    f32 = jnp.float32
    src = edge_index[0].astype(jnp.int32)
    dst = edge_index[1].astype(jnp.int32)
    ew = edge_weight.astype(f32)

    xp = jnp.pad(x.astype(f32), ((0, NPAD - N), (0, 0)))
    srcp = jnp.concatenate([src, jnp.full((EPAD - E,), N, jnp.int32)])
    dstp = jnp.concatenate([dst, jnp.full((EPAD - E,), N, jnp.int32)])
    ewp = jnp.concatenate([ew, jnp.zeros((EPAD - E,), f32)])
    src2d = srcp.reshape(EPAD // ECH, ECH)
    dst2d = dstp.reshape(EPAD // ECH, ECH)
    ew2d = ewp.reshape(EPAD // ECH, ECH)
    ew16 = jnp.pad(ewp[:, None], ((0, 0), (0, DEGW - 1)))

    b1 = gcn1_b.astype(f32).reshape(1, D)
    b2 = gcn2_b.astype(f32).reshape(1, C)

    degp = _sc_degree(ew16, dst2d)

    xw1, y1 = pl.pallas_call(
        _tc_xw_y,
        grid=(_GRID,),
        in_specs=[_row_spec(D), _full_spec(D, D), _part_spec(DEGW)],
        out_specs=[_row_spec(D), _row_spec(D)],
        out_shape=[jax.ShapeDtypeStruct((NPAD, D), f32)] * 2,
    )(xp, gcn1_W.astype(f32), degp)

    agg1 = _sc_edge_pass(D)(y1, src2d, dst2d, ew2d)

    xw2, y2 = pl.pallas_call(
        _tc_mid,
        grid=(_GRID,),
        in_specs=[_part_spec(D), _row_spec(D), _part_spec(DEGW),
                  _full_spec(1, D), _full_spec(D, C)],
        out_specs=[_row_spec(C), _row_spec(C)],
        out_shape=[jax.ShapeDtypeStruct((NPAD, C), f32)] * 2,
    )(agg1, xw1, degp, b1, gcn2_W.astype(f32))

    agg2 = _sc_edge_pass(C)(y2, src2d, dst2d, ew2d)

    out = pl.pallas_call(
        _tc_final,
        grid=(_GRID,),
        in_specs=[_part_spec(C), _row_spec(C), _part_spec(DEGW),
                  _full_spec(1, C)],
        out_specs=_row_spec(C),
        out_shape=jax.ShapeDtypeStruct((NPAD, C), f32),
    )(agg2, xw2, degp, b2)

    return out[:N]


# speedup vs baseline: 6.3944x; 6.3944x over previous
import functools

import jax
import jax.numpy as jnp
from jax import lax
from jax.experimental import pallas as pl
from jax.experimental.pallas import tpu as pltpu
from jax.experimental.pallas import tpu_sc as plsc

N = 10000
D = 128
C = 32

NC = 2
NS = 16
NW = NC * NS
LANES = 16

NPAD = 10240
E = 160000
EPAD = 163840
ECH = 128
EPT = EPAD // NW
NCH = EPT // ECH
ROWS_PT = NPAD // NS
DEGW = 128

_mesh = functools.partial(plsc.VectorSubcoreMesh,
                          core_axis_name="c", subcore_axis_name="s")


def _sc_edge_pass(F):
    @functools.partial(
        pl.kernel,
        out_type=jax.ShapeDtypeStruct((NC, NPAD, F), jnp.float32),
        mesh=_mesh(),
        scratch_types=[
            pltpu.VMEM((NCH, ECH), jnp.int32),
            pltpu.VMEM((NCH, ECH), jnp.int32),
            pltpu.VMEM((NCH, ECH), jnp.float32),
            pltpu.VMEM((ECH, F), jnp.float32),
            pltpu.VMEM_SHARED((NPAD, F), jnp.float32),
            pltpu.SemaphoreType.DMA,
        ],
    )
    def k(y_hbm, src_hbm, dst_hbm, ew_hbm, out_hbm,
          src_v, dst_v, ew_v, rows_v, acc_sh, sem):
        cid = lax.axis_index("c")
        sid = lax.axis_index("s")
        t = cid * NS + sid
        ebase = t * NCH
        pltpu.sync_copy(src_hbm.at[pl.ds(ebase, NCH)], src_v)
        pltpu.sync_copy(dst_hbm.at[pl.ds(ebase, NCH)], dst_v)
        pltpu.sync_copy(ew_hbm.at[pl.ds(ebase, NCH)], ew_v)

        zv = jnp.zeros((LANES,), jnp.float32)

        def zrow(r, carry):
            for cb in range(F // LANES):
                rows_v[r, pl.ds(cb * LANES, LANES)] = zv
            return carry

        lax.fori_loop(0, ECH, zrow, 0)
        rbase = sid * ROWS_PT
        for rb in range(ROWS_PT // ECH):
            pltpu.sync_copy(rows_v, acc_sh.at[pl.ds(rbase + rb * ECH, ECH)])
        plsc.subcore_barrier()

        def chunk(j, carry):
            pltpu.async_copy(y_hbm.at[src_v.at[j]], rows_v, sem).wait()

            def grp(kk, c2):
                w16 = ew_v[j, pl.ds(kk * LANES, LANES)]
                for l in range(LANES):
                    w = jnp.broadcast_to(w16[l], (LANES,))
                    e = kk * LANES + l
                    for cb in range(F // LANES):
                        sl = pl.ds(cb * LANES, LANES)
                        rows_v[e, sl] = rows_v[e, sl] * w
                return c2

            lax.fori_loop(0, ECH // LANES, grp, 0)
            pltpu.sync_copy(rows_v, acc_sh.at[dst_v.at[j]], add=True)
            return carry

        lax.fori_loop(0, NCH, chunk, 0)
        plsc.subcore_barrier()

        for rb in range(ROWS_PT // ECH):
            sl = pl.ds(rbase + rb * ECH, ECH)
            pltpu.sync_copy(acc_sh.at[sl], rows_v)
            pltpu.sync_copy(rows_v, out_hbm.at[cid, sl])

    return k


@functools.partial(
    pl.kernel,
    out_type=jax.ShapeDtypeStruct((NC, NPAD, DEGW), jnp.float32),
    mesh=_mesh(),
    scratch_types=[
        pltpu.VMEM((NCH, ECH), jnp.int32),
        pltpu.VMEM((NCH, ECH), jnp.float32),
        pltpu.VMEM((ECH, DEGW), jnp.float32),
        pltpu.VMEM_SHARED((NPAD, DEGW), jnp.float32),
    ],
)
def _sc_degree(ew_hbm, dst_hbm, out_hbm, dst_v, ew_v, rows_v, acc_sh):
    cid = lax.axis_index("c")
    sid = lax.axis_index("s")
    t = cid * NS + sid
    pltpu.sync_copy(dst_hbm.at[pl.ds(t * NCH, NCH)], dst_v)
    pltpu.sync_copy(ew_hbm.at[pl.ds(t * NCH, NCH)], ew_v)

    zv = jnp.zeros((LANES,), jnp.float32)

    def zrow(r, carry):
        for cb in range(DEGW // LANES):
            rows_v[r, pl.ds(cb * LANES, LANES)] = zv
        return carry

    lax.fori_loop(0, ECH, zrow, 0)
    rbase = sid * ROWS_PT
    for rb in range(ROWS_PT // ECH):
        pltpu.sync_copy(rows_v, acc_sh.at[pl.ds(rbase + rb * ECH, ECH)])
    plsc.subcore_barrier()

    def chunk(j, carry):
        def grp(kk, c2):
            w16 = ew_v[j, pl.ds(kk * LANES, LANES)]
            for l in range(LANES):
                w = jnp.broadcast_to(w16[l], (LANES,))
                e = kk * LANES + l
                for cb in range(DEGW // LANES):
                    rows_v[e, pl.ds(cb * LANES, LANES)] = w
            return c2

        lax.fori_loop(0, ECH // LANES, grp, 0)
        pltpu.sync_copy(rows_v, acc_sh.at[dst_v.at[j]], add=True)
        return carry

    lax.fori_loop(0, NCH, chunk, 0)
    plsc.subcore_barrier()

    for rb in range(ROWS_PT // ECH):
        sl = pl.ds(rbase + rb * ECH, ECH)
        pltpu.sync_copy(acc_sh.at[sl], rows_v)
        pltpu.sync_copy(rows_v, out_hbm.at[cid, sl])


RB = 1280


def _deg_stats(degp_blk):
    deg = 1.0 + degp_blk[0, :, 0] + degp_blk[1, :, 0]
    return lax.rsqrt(deg), 1.0 / deg


def _tc_xw_y(x_ref, w_ref, degp_ref, xw_ref, y_ref):
    dis, _ = _deg_stats(degp_ref[...])
    xw = jnp.dot(x_ref[...], w_ref[...], preferred_element_type=jnp.float32)
    xw_ref[...] = xw
    y_ref[...] = xw * dis[:, None]


def _tc_mid(a_ref, xw1_ref, degp_ref, b1_ref, w2_ref, xw2_ref, z_ref):
    dis, inv = _deg_stats(degp_ref[...])
    a = a_ref[...]
    h = dis[:, None] * (a[0] + a[1]) + inv[:, None] * xw1_ref[...] + b1_ref[...]
    h = jnp.maximum(h, 0.0)
    xw2_ref[...] = jnp.dot(h, w2_ref[...], preferred_element_type=jnp.float32)
    z_ref[...] = h * dis[:, None]


def _tc_final(a_ref, xw2_ref, degp_ref, b2_ref, w2_ref, o_ref):
    dis, inv = _deg_stats(degp_ref[...])
    a = a_ref[...]
    agg2 = jnp.dot(a[0] + a[1], w2_ref[...], preferred_element_type=jnp.float32)
    s = dis[:, None] * agg2 + inv[:, None] * xw2_ref[...] + b2_ref[...]
    m = jnp.max(s, axis=-1, keepdims=True)
    ex = jnp.exp(s - m)
    o_ref[...] = ex / jnp.sum(ex, axis=-1, keepdims=True)


def _row_spec(f):
    return pl.BlockSpec((RB, f), lambda i: (i, 0))


def _part_spec(f):
    return pl.BlockSpec((NC, RB, f), lambda i: (0, i, 0))


def _full_spec(r, f):
    return pl.BlockSpec((r, f), lambda i: (0, 0))


_GRID = NPAD // RB


def kernel(x, edge_index, edge_weight, gat_W, gat_att_src, gat_att_dst,
           gat_bias, gcn1_W, gcn1_b, gcn2_W, gcn2_b):
    f32 = jnp.float32
    src = edge_index[0].astype(jnp.int32)
    dst = edge_index[1].astype(jnp.int32)
    ew = edge_weight.astype(f32)

    xp = jnp.pad(x.astype(f32), ((0, NPAD - N), (0, 0)))
    srcp = jnp.concatenate([src, jnp.full((EPAD - E,), N, jnp.int32)])
    dstp = jnp.concatenate([dst, jnp.full((EPAD - E,), N, jnp.int32)])
    ewp = jnp.concatenate([ew, jnp.zeros((EPAD - E,), f32)])
    src2d = srcp.reshape(EPAD // ECH, ECH)
    dst2d = dstp.reshape(EPAD // ECH, ECH)
    ew2d = ewp.reshape(EPAD // ECH, ECH)

    b1 = gcn1_b.astype(f32).reshape(1, D)
    b2 = gcn2_b.astype(f32).reshape(1, C)

    degp = _sc_degree(ew2d, dst2d)

    xw1, y1 = pl.pallas_call(
        _tc_xw_y,
        grid=(_GRID,),
        in_specs=[_row_spec(D), _full_spec(D, D), _part_spec(DEGW)],
        out_specs=[_row_spec(D), _row_spec(D)],
        out_shape=[jax.ShapeDtypeStruct((NPAD, D), f32)] * 2,
    )(xp, gcn1_W.astype(f32), degp)

    agg1 = _sc_edge_pass(D)(y1, src2d, dst2d, ew2d)

    w2 = gcn2_W.astype(f32)
    xw2, z = pl.pallas_call(
        _tc_mid,
        grid=(_GRID,),
        in_specs=[_part_spec(D), _row_spec(D), _part_spec(DEGW),
                  _full_spec(1, D), _full_spec(D, C)],
        out_specs=[_row_spec(C), _row_spec(D)],
        out_shape=[jax.ShapeDtypeStruct((NPAD, C), f32),
                   jax.ShapeDtypeStruct((NPAD, D), f32)],
    )(agg1, xw1, degp, b1, w2)

    aggz = _sc_edge_pass(D)(z, src2d, dst2d, ew2d)

    out = pl.pallas_call(
        _tc_final,
        grid=(_GRID,),
        in_specs=[_part_spec(D), _row_spec(C), _part_spec(DEGW),
                  _full_spec(1, C), _full_spec(D, C)],
        out_specs=_row_spec(C),
        out_shape=jax.ShapeDtypeStruct((NPAD, C), f32),
    )(aggz, xw2, degp, b2, w2)

    return out[:N]


# speedup vs baseline: 6.8398x; 1.0696x over previous
import functools

import jax
import jax.numpy as jnp
from jax import lax
from jax.experimental import pallas as pl
from jax.experimental.pallas import tpu as pltpu
from jax.experimental.pallas import tpu_sc as plsc

N = 10000
D = 128
C = 32

NC = 2
NS = 16
NW = NC * NS
LANES = 16

NPAD = 10240
E = 160000
EPAD = 163840
ECH = 128
EPT = EPAD // NW
NCH = EPT // ECH
ROWS_PT = NPAD // NS
DEGW = 128

_mesh = functools.partial(plsc.VectorSubcoreMesh,
                          core_axis_name="c", subcore_axis_name="s")


def _sc_edge_pass(F):
    @functools.partial(
        pl.kernel,
        out_type=jax.ShapeDtypeStruct((NC, NPAD, F), jnp.float32),
        mesh=_mesh(),
        scratch_types=[
            pltpu.VMEM((NCH, ECH), jnp.int32),
            pltpu.VMEM((NCH, ECH), jnp.int32),
            pltpu.VMEM((NCH, ECH), jnp.float32),
            pltpu.VMEM((ECH, F), jnp.float32),
            pltpu.VMEM((ECH, F), jnp.float32),
            pltpu.VMEM_SHARED((NPAD, F), jnp.float32),
            pltpu.SemaphoreType.DMA,
            pltpu.SemaphoreType.DMA,
            pltpu.SemaphoreType.DMA,
        ],
    )
    def k(y_hbm, src_hbm, dst_hbm, ew_hbm, out_hbm,
          src_v, dst_v, ew_v, rows_a, rows_b, acc_sh, gsa, gsb, ssem):
        cid = lax.axis_index("c")
        sid = lax.axis_index("s")
        t = cid * NS + sid
        ebase = t * NCH
        pltpu.sync_copy(src_hbm.at[pl.ds(ebase, NCH)], src_v)
        pltpu.sync_copy(dst_hbm.at[pl.ds(ebase, NCH)], dst_v)
        pltpu.sync_copy(ew_hbm.at[pl.ds(ebase, NCH)], ew_v)

        zv = jnp.zeros((LANES,), jnp.float32)

        def zrow(r, carry):
            for cb in range(F // LANES):
                rows_a[r, pl.ds(cb * LANES, LANES)] = zv
            return carry

        lax.fori_loop(0, ECH, zrow, 0)
        rbase = sid * ROWS_PT
        for rb in range(ROWS_PT // ECH):
            pltpu.sync_copy(rows_a, acc_sh.at[pl.ds(rbase + rb * ECH, ECH)])
        plsc.subcore_barrier()

        def scale(buf, j):
            def grp(kk, c2):
                w16 = ew_v[j, pl.ds(kk * LANES, LANES)]
                for l in range(LANES):
                    w = jnp.broadcast_to(w16[l], (LANES,))
                    e = kk * LANES + l
                    for cb in range(F // LANES):
                        sl = pl.ds(cb * LANES, LANES)
                        buf[e, sl] = buf[e, sl] * w
                return c2

            lax.fori_loop(0, ECH // LANES, grp, 0)

        pltpu.async_copy(y_hbm.at[src_v.at[0]], rows_a, gsa)
        pltpu.async_copy(y_hbm.at[src_v.at[1]], rows_b, gsb)

        def pair(p, carry):
            j0 = 2 * p
            j1 = j0 + 1
            pltpu.make_async_copy(y_hbm.at[src_v.at[j0]], rows_a, gsa).wait()
            scale(rows_a, j0)
            pltpu.async_copy(rows_a, acc_sh.at[dst_v.at[j0]], ssem, add=True)
            pltpu.make_async_copy(y_hbm.at[src_v.at[j1]], rows_b, gsb).wait()
            scale(rows_b, j1)
            pltpu.async_copy(rows_b, acc_sh.at[dst_v.at[j1]], ssem, add=True)
            jn0 = jnp.minimum(j0 + 2, NCH - 2)
            jn1 = jnp.minimum(j1 + 2, NCH - 1)
            pltpu.make_async_copy(rows_a, acc_sh.at[dst_v.at[j0]], ssem).wait()
            pltpu.async_copy(y_hbm.at[src_v.at[jn0]], rows_a, gsa)
            pltpu.make_async_copy(rows_b, acc_sh.at[dst_v.at[j1]], ssem).wait()
            pltpu.async_copy(y_hbm.at[src_v.at[jn1]], rows_b, gsb)
            return carry

        lax.fori_loop(0, NCH // 2, pair, 0)
        pltpu.make_async_copy(y_hbm.at[src_v.at[NCH - 2]], rows_a, gsa).wait()
        pltpu.make_async_copy(y_hbm.at[src_v.at[NCH - 1]], rows_b, gsb).wait()
        plsc.subcore_barrier()

        for rb in range(ROWS_PT // ECH):
            sl = pl.ds(rbase + rb * ECH, ECH)
            pltpu.sync_copy(acc_sh.at[sl], rows_a)
            pltpu.sync_copy(rows_a, out_hbm.at[cid, sl])

    return k


@functools.partial(
    pl.kernel,
    out_type=jax.ShapeDtypeStruct((NC, NPAD, DEGW), jnp.float32),
    mesh=_mesh(),
    scratch_types=[
        pltpu.VMEM((NCH, ECH), jnp.int32),
        pltpu.VMEM((NCH, ECH), jnp.float32),
        pltpu.VMEM((ECH, DEGW), jnp.float32),
        pltpu.VMEM_SHARED((NPAD, DEGW), jnp.float32),
    ],
)
def _sc_degree(ew_hbm, dst_hbm, out_hbm, dst_v, ew_v, rows_v, acc_sh):
    cid = lax.axis_index("c")
    sid = lax.axis_index("s")
    t = cid * NS + sid
    pltpu.sync_copy(dst_hbm.at[pl.ds(t * NCH, NCH)], dst_v)
    pltpu.sync_copy(ew_hbm.at[pl.ds(t * NCH, NCH)], ew_v)

    zv = jnp.zeros((LANES,), jnp.float32)

    def zrow(r, carry):
        for cb in range(DEGW // LANES):
            rows_v[r, pl.ds(cb * LANES, LANES)] = zv
        return carry

    lax.fori_loop(0, ECH, zrow, 0)
    rbase = sid * ROWS_PT
    for rb in range(ROWS_PT // ECH):
        pltpu.sync_copy(rows_v, acc_sh.at[pl.ds(rbase + rb * ECH, ECH)])
    plsc.subcore_barrier()

    def chunk(j, carry):
        def grp(kk, c2):
            w16 = ew_v[j, pl.ds(kk * LANES, LANES)]
            for l in range(LANES):
                w = jnp.broadcast_to(w16[l], (LANES,))
                e = kk * LANES + l
                for cb in range(DEGW // LANES):
                    rows_v[e, pl.ds(cb * LANES, LANES)] = w
            return c2

        lax.fori_loop(0, ECH // LANES, grp, 0)
        pltpu.sync_copy(rows_v, acc_sh.at[dst_v.at[j]], add=True)
        return carry

    lax.fori_loop(0, NCH, chunk, 0)
    plsc.subcore_barrier()

    for rb in range(ROWS_PT // ECH):
        sl = pl.ds(rbase + rb * ECH, ECH)
        pltpu.sync_copy(acc_sh.at[sl], rows_v)
        pltpu.sync_copy(rows_v, out_hbm.at[cid, sl])


RB = 1280


def _deg_stats(degp_blk):
    deg = 1.0 + degp_blk[0, :, 0] + degp_blk[1, :, 0]
    return lax.rsqrt(deg), 1.0 / deg


def _tc_xw_y(x_ref, w_ref, degp_ref, xw_ref, y_ref):
    dis, _ = _deg_stats(degp_ref[...])
    xw = jnp.dot(x_ref[...], w_ref[...], preferred_element_type=jnp.float32)
    xw_ref[...] = xw
    y_ref[...] = xw * dis[:, None]


def _tc_mid(a_ref, xw1_ref, degp_ref, b1_ref, w2_ref, xw2_ref, z_ref):
    dis, inv = _deg_stats(degp_ref[...])
    a = a_ref[...]
    h = dis[:, None] * (a[0] + a[1]) + inv[:, None] * xw1_ref[...] + b1_ref[...]
    h = jnp.maximum(h, 0.0)
    xw2_ref[...] = jnp.dot(h, w2_ref[...], preferred_element_type=jnp.float32)
    z_ref[...] = h * dis[:, None]


def _tc_final(a_ref, xw2_ref, degp_ref, b2_ref, w2_ref, o_ref):
    dis, inv = _deg_stats(degp_ref[...])
    a = a_ref[...]
    agg2 = jnp.dot(a[0] + a[1], w2_ref[...], preferred_element_type=jnp.float32)
    s = dis[:, None] * agg2 + inv[:, None] * xw2_ref[...] + b2_ref[...]
    m = jnp.max(s, axis=-1, keepdims=True)
    ex = jnp.exp(s - m)
    o_ref[...] = ex / jnp.sum(ex, axis=-1, keepdims=True)


def _row_spec(f):
    return pl.BlockSpec((RB, f), lambda i: (i, 0))


def _part_spec(f):
    return pl.BlockSpec((NC, RB, f), lambda i: (0, i, 0))


def _full_spec(r, f):
    return pl.BlockSpec((r, f), lambda i: (0, 0))


_GRID = NPAD // RB


def kernel(x, edge_index, edge_weight, gat_W, gat_att_src, gat_att_dst,
           gat_bias, gcn1_W, gcn1_b, gcn2_W, gcn2_b):
    f32 = jnp.float32
    src = edge_index[0].astype(jnp.int32)
    dst = edge_index[1].astype(jnp.int32)
    ew = edge_weight.astype(f32)

    xp = jnp.pad(x.astype(f32), ((0, NPAD - N), (0, 0)))
    srcp = jnp.concatenate([src, jnp.full((EPAD - E,), N, jnp.int32)])
    dstp = jnp.concatenate([dst, jnp.full((EPAD - E,), N, jnp.int32)])
    ewp = jnp.concatenate([ew, jnp.zeros((EPAD - E,), f32)])
    src2d = srcp.reshape(EPAD // ECH, ECH)
    dst2d = dstp.reshape(EPAD // ECH, ECH)
    ew2d = ewp.reshape(EPAD // ECH, ECH)

    b1 = gcn1_b.astype(f32).reshape(1, D)
    b2 = gcn2_b.astype(f32).reshape(1, C)

    degp = _sc_degree(ew2d, dst2d)

    xw1, y1 = pl.pallas_call(
        _tc_xw_y,
        grid=(_GRID,),
        in_specs=[_row_spec(D), _full_spec(D, D), _part_spec(DEGW)],
        out_specs=[_row_spec(D), _row_spec(D)],
        out_shape=[jax.ShapeDtypeStruct((NPAD, D), f32)] * 2,
    )(xp, gcn1_W.astype(f32), degp)

    agg1 = _sc_edge_pass(D)(y1, src2d, dst2d, ew2d)

    w2 = gcn2_W.astype(f32)
    xw2, z = pl.pallas_call(
        _tc_mid,
        grid=(_GRID,),
        in_specs=[_part_spec(D), _row_spec(D), _part_spec(DEGW),
                  _full_spec(1, D), _full_spec(D, C)],
        out_specs=[_row_spec(C), _row_spec(D)],
        out_shape=[jax.ShapeDtypeStruct((NPAD, C), f32),
                   jax.ShapeDtypeStruct((NPAD, D), f32)],
    )(agg1, xw1, degp, b1, w2)

    aggz = _sc_edge_pass(D)(z, src2d, dst2d, ew2d)

    out = pl.pallas_call(
        _tc_final,
        grid=(_GRID,),
        in_specs=[_part_spec(D), _row_spec(C), _part_spec(DEGW),
                  _full_spec(1, C), _full_spec(D, C)],
        out_specs=_row_spec(C),
        out_shape=jax.ShapeDtypeStruct((NPAD, C), f32),
    )(aggz, xw2, degp, b2, w2)

    return out[:N]


# speedup vs baseline: 7.1020x; 1.0383x over previous
import functools

import jax
import jax.numpy as jnp
from jax import lax
from jax.experimental import pallas as pl
from jax.experimental.pallas import tpu as pltpu
from jax.experimental.pallas import tpu_sc as plsc

N = 10000
D = 128
C = 32

NC = 2
NS = 16
NW = NC * NS
LANES = 16

NPAD = 10240
E = 160000
EPAD = 163840
ECH = 128
EPT = EPAD // NW
NCH = EPT // ECH
ROWS_PT = NPAD // NS
DEGW = 128

_mesh = functools.partial(plsc.VectorSubcoreMesh,
                          core_axis_name="c", subcore_axis_name="s")


def _sc_edge_pass(F, active_blocks):
    @functools.partial(
        pl.kernel,
        out_type=jax.ShapeDtypeStruct((NC, NPAD, F), jnp.float32),
        mesh=_mesh(),
        scratch_types=[
            pltpu.VMEM((NCH, ECH), jnp.int32),
            pltpu.VMEM((NCH, ECH), jnp.int32),
            pltpu.VMEM((NCH, ECH), jnp.float32),
            pltpu.VMEM((ECH, F), jnp.float32),
            pltpu.VMEM((ECH, F), jnp.float32),
            pltpu.VMEM_SHARED((NPAD, F), jnp.float32),
            pltpu.SemaphoreType.DMA,
            pltpu.SemaphoreType.DMA,
            pltpu.SemaphoreType.DMA,
        ],
    )
    def k(y_hbm, src_hbm, dst_hbm, ew_hbm, out_hbm,
          src_v, dst_v, ew_v, rows_a, rows_b, acc_sh, gsa, gsb, ssem):
        cid = lax.axis_index("c")
        sid = lax.axis_index("s")
        t = cid * NS + sid
        ebase = t * NCH
        pltpu.sync_copy(src_hbm.at[pl.ds(ebase, NCH)], src_v)
        pltpu.sync_copy(dst_hbm.at[pl.ds(ebase, NCH)], dst_v)
        pltpu.sync_copy(ew_hbm.at[pl.ds(ebase, NCH)], ew_v)

        zv = jnp.zeros((LANES,), jnp.float32)

        def zrow(r, carry):
            for cb in range(F // LANES):
                rows_a[r, pl.ds(cb * LANES, LANES)] = zv
            return carry

        lax.fori_loop(0, ECH, zrow, 0)
        rbase = sid * ROWS_PT
        for rb in range(ROWS_PT // ECH):
            pltpu.sync_copy(rows_a, acc_sh.at[pl.ds(rbase + rb * ECH, ECH)])
        plsc.subcore_barrier()

        def scale(buf, j):
            def grp(kk, c2):
                w16 = ew_v[j, pl.ds(kk * LANES, LANES)]
                for l in range(LANES):
                    w = jnp.broadcast_to(w16[l], (LANES,))
                    e = kk * LANES + l
                    for cb in range(active_blocks):
                        sl = pl.ds(cb * LANES, LANES)
                        buf[e, sl] = buf[e, sl] * w
                return c2

            lax.fori_loop(0, ECH // LANES, grp, 0)

        pltpu.async_copy(y_hbm.at[src_v.at[0]], rows_a, gsa)
        pltpu.async_copy(y_hbm.at[src_v.at[1]], rows_b, gsb)

        def pair(p, carry):
            j0 = 2 * p
            j1 = j0 + 1
            pltpu.make_async_copy(y_hbm.at[src_v.at[j0]], rows_a, gsa).wait()
            scale(rows_a, j0)
            pltpu.async_copy(rows_a, acc_sh.at[dst_v.at[j0]], ssem, add=True)
            pltpu.make_async_copy(y_hbm.at[src_v.at[j1]], rows_b, gsb).wait()
            scale(rows_b, j1)
            pltpu.async_copy(rows_b, acc_sh.at[dst_v.at[j1]], ssem, add=True)
            jn0 = jnp.minimum(j0 + 2, NCH - 2)
            jn1 = jnp.minimum(j1 + 2, NCH - 1)
            pltpu.make_async_copy(rows_a, acc_sh.at[dst_v.at[j0]], ssem).wait()
            pltpu.async_copy(y_hbm.at[src_v.at[jn0]], rows_a, gsa)
            pltpu.make_async_copy(rows_b, acc_sh.at[dst_v.at[j1]], ssem).wait()
            pltpu.async_copy(y_hbm.at[src_v.at[jn1]], rows_b, gsb)
            return carry

        lax.fori_loop(0, NCH // 2, pair, 0)
        pltpu.make_async_copy(y_hbm.at[src_v.at[NCH - 2]], rows_a, gsa).wait()
        pltpu.make_async_copy(y_hbm.at[src_v.at[NCH - 1]], rows_b, gsb).wait()
        plsc.subcore_barrier()

        for rb in range(ROWS_PT // ECH):
            sl = pl.ds(rbase + rb * ECH, ECH)
            pltpu.sync_copy(acc_sh.at[sl], rows_a)
            pltpu.sync_copy(rows_a, out_hbm.at[cid, sl])

    return k


@functools.partial(
    pl.kernel,
    out_type=jax.ShapeDtypeStruct((NC, NPAD, DEGW), jnp.float32),
    mesh=_mesh(),
    scratch_types=[
        pltpu.VMEM((NCH, ECH), jnp.int32),
        pltpu.VMEM((NCH, ECH), jnp.float32),
        pltpu.VMEM((ECH, DEGW), jnp.float32),
        pltpu.VMEM_SHARED((NPAD, DEGW), jnp.float32),
    ],
)
def _sc_degree(ew_hbm, dst_hbm, out_hbm, dst_v, ew_v, rows_v, acc_sh):
    cid = lax.axis_index("c")
    sid = lax.axis_index("s")
    t = cid * NS + sid
    pltpu.sync_copy(dst_hbm.at[pl.ds(t * NCH, NCH)], dst_v)
    pltpu.sync_copy(ew_hbm.at[pl.ds(t * NCH, NCH)], ew_v)

    zv = jnp.zeros((LANES,), jnp.float32)

    def zrow(r, carry):
        for cb in range(DEGW // LANES):
            rows_v[r, pl.ds(cb * LANES, LANES)] = zv
        return carry

    lax.fori_loop(0, ECH, zrow, 0)
    rbase = sid * ROWS_PT
    for rb in range(ROWS_PT // ECH):
        pltpu.sync_copy(rows_v, acc_sh.at[pl.ds(rbase + rb * ECH, ECH)])
    plsc.subcore_barrier()

    def chunk(j, carry):
        def grp(kk, c2):
            w16 = ew_v[j, pl.ds(kk * LANES, LANES)]
            for l in range(LANES):
                w = jnp.broadcast_to(w16[l], (LANES,))
                e = kk * LANES + l
                rows_v[e, pl.ds(0, LANES)] = w
            return c2

        lax.fori_loop(0, ECH // LANES, grp, 0)
        pltpu.sync_copy(rows_v, acc_sh.at[dst_v.at[j]], add=True)
        return carry

    lax.fori_loop(0, NCH, chunk, 0)
    plsc.subcore_barrier()

    for rb in range(ROWS_PT // ECH):
        sl = pl.ds(rbase + rb * ECH, ECH)
        pltpu.sync_copy(acc_sh.at[sl], rows_v)
        pltpu.sync_copy(rows_v, out_hbm.at[cid, sl])


RB = 1280


def _deg_stats(degp_blk):
    deg = 1.0 + degp_blk[0, :, 0] + degp_blk[1, :, 0]
    return lax.rsqrt(deg), 1.0 / deg


def _tc_xw_y(x_ref, w_ref, degp_ref, xw_ref, y_ref):
    dis, _ = _deg_stats(degp_ref[...])
    xw = jnp.dot(x_ref[...], w_ref[...], preferred_element_type=jnp.float32)
    xw_ref[...] = xw
    y_ref[...] = xw * dis[:, None]


def _tc_mid(a_ref, xw1_ref, degp_ref, b1_ref, w2_ref, xw2_ref, z_ref):
    dis, inv = _deg_stats(degp_ref[...])
    a = a_ref[...]
    h = dis[:, None] * (a[0] + a[1]) + inv[:, None] * xw1_ref[...] + b1_ref[...]
    h = jnp.maximum(h, 0.0)
    xw2 = jnp.dot(h, w2_ref[...], preferred_element_type=jnp.float32)
    xw2_ref[...] = xw2
    z_ref[...] = jnp.concatenate(
        [xw2 * dis[:, None], jnp.zeros((RB, D - C), jnp.float32)], axis=1)


def _tc_final(a_ref, xw2_ref, degp_ref, b2_ref, o_ref):
    dis, inv = _deg_stats(degp_ref[...])
    a = a_ref[...]
    agg2 = (a[0] + a[1])[:, :C]
    s = dis[:, None] * agg2 + inv[:, None] * xw2_ref[...] + b2_ref[...]
    m = jnp.max(s, axis=-1, keepdims=True)
    ex = jnp.exp(s - m)
    o_ref[...] = ex / jnp.sum(ex, axis=-1, keepdims=True)


def _row_spec(f):
    return pl.BlockSpec((RB, f), lambda i: (i, 0))


def _part_spec(f):
    return pl.BlockSpec((NC, RB, f), lambda i: (0, i, 0))


def _full_spec(r, f):
    return pl.BlockSpec((r, f), lambda i: (0, 0))


_GRID = NPAD // RB


def kernel(x, edge_index, edge_weight, gat_W, gat_att_src, gat_att_dst,
           gat_bias, gcn1_W, gcn1_b, gcn2_W, gcn2_b):
    f32 = jnp.float32
    src = edge_index[0].astype(jnp.int32)
    dst = edge_index[1].astype(jnp.int32)
    ew = edge_weight.astype(f32)

    xp = jnp.pad(x.astype(f32), ((0, NPAD - N), (0, 0)))
    srcp = jnp.concatenate([src, jnp.full((EPAD - E,), N, jnp.int32)])
    dstp = jnp.concatenate([dst, jnp.full((EPAD - E,), N, jnp.int32)])
    ewp = jnp.concatenate([ew, jnp.zeros((EPAD - E,), f32)])
    src2d = srcp.reshape(EPAD // ECH, ECH)
    dst2d = dstp.reshape(EPAD // ECH, ECH)
    ew2d = ewp.reshape(EPAD // ECH, ECH)

    b1 = gcn1_b.astype(f32).reshape(1, D)
    b2 = gcn2_b.astype(f32).reshape(1, C)

    degp = _sc_degree(ew2d, dst2d)

    xw1, y1 = pl.pallas_call(
        _tc_xw_y,
        grid=(_GRID,),
        in_specs=[_row_spec(D), _full_spec(D, D), _part_spec(DEGW)],
        out_specs=[_row_spec(D), _row_spec(D)],
        out_shape=[jax.ShapeDtypeStruct((NPAD, D), f32)] * 2,
    )(xp, gcn1_W.astype(f32), degp)

    agg1 = _sc_edge_pass(D, D // LANES)(y1, src2d, dst2d, ew2d)

    w2 = gcn2_W.astype(f32)
    xw2, z = pl.pallas_call(
        _tc_mid,
        grid=(_GRID,),
        in_specs=[_part_spec(D), _row_spec(D), _part_spec(DEGW),
                  _full_spec(1, D), _full_spec(D, C)],
        out_specs=[_row_spec(C), _row_spec(D)],
        out_shape=[jax.ShapeDtypeStruct((NPAD, C), f32),
                   jax.ShapeDtypeStruct((NPAD, D), f32)],
    )(agg1, xw1, degp, b1, w2)

    aggz = _sc_edge_pass(D, C // LANES)(z, src2d, dst2d, ew2d)

    out = pl.pallas_call(
        _tc_final,
        grid=(_GRID,),
        in_specs=[_part_spec(D), _row_spec(C), _part_spec(DEGW),
                  _full_spec(1, C)],
        out_specs=_row_spec(C),
        out_shape=jax.ShapeDtypeStruct((NPAD, C), f32),
    )(aggz, xw2, degp, b2)

    return out[:N]
